# baseline (speedup 1.0000x reference)
import functools

import jax
import jax.numpy as jnp
from jax import lax
from jax.experimental import pallas as pl
from jax.experimental.pallas import tpu as pltpu
from jax.experimental.pallas import tpu_sc as plsc

_NC = 2
_NS = 16
_NW = _NC * _NS
_BB = 80
_CH = 16
_NK = 4


def _make_sc_scatter(n_rows, d, acc_rows, nb, with_count):
  rows_per_tile = acc_rows // _NS
  mesh = plsc.VectorSubcoreMesh(core_axis_name="c", subcore_axis_name="s")
  outs = [jax.ShapeDtypeStruct((_NC * acc_rows, d), jnp.float32)]
  scratch = [
      pltpu.VMEM((_CH, _BB), jnp.int32),
      pltpu.VMEM((_CH, _BB), jnp.int32),
  ] + [
      pltpu.VMEM((_BB, d), jnp.float32)
      for _ in range(_NK)
  ] + [
      pltpu.VMEM((8, d), jnp.float32),
      pltpu.VMEM_SHARED((acc_rows, d), jnp.float32),
      pltpu.SemaphoreType.DMA,
      pltpu.SemaphoreType.DMA,
  ]
  if with_count:
    outs.append(jax.ShapeDtypeStruct((_NC * acc_rows, d), jnp.float32))

  def body(h_hbm, src_hbm, dst_hbm, *rest):
    if with_count:
      sum_hbm, cnt_hbm, srcv, dstv = rest[:4]
      bufs = rest[4:4 + _NK]
      zbuf, acc, gsem, ssem = rest[4 + _NK:]
    else:
      sum_hbm, srcv, dstv = rest[:3]
      bufs = rest[3:3 + _NK]
      zbuf, acc, gsem, ssem = rest[3 + _NK:]
    buf0 = bufs[0]
    cid = lax.axis_index("c")
    sid = lax.axis_index("s")
    wid = sid * _NC + cid
    zv = jnp.zeros((16,), jnp.float32)
    for r in range(8):
      for c8 in range(d // 16):
        zbuf[r, pl.ds(c8 * 16, 16)] = zv
    base = sid * rows_per_tile
    obase = cid * acc_rows + base

    def zero_acc():
      def zstep(j, c):
        pltpu.sync_copy(zbuf, acc.at[pl.ds(base + j * 8, 8)])
        return c
      lax.fori_loop(0, rows_per_tile // 8, zstep, 0)

    zero_acc()
    if with_count:
      ov = jnp.ones((16,), jnp.float32)
      for r in range(_BB):
        for c8 in range(d // 16):
          buf0[r, pl.ds(c8 * 16, 16)] = ov
      plsc.subcore_barrier()

      def cstep(k, c):
        row0 = wid * nb + k * _CH
        pltpu.sync_copy(dst_hbm.at[pl.ds(row0, _CH)], dstv)
        copies = [pltpu.async_copy(buf0, acc.at[dstv.at[j]], ssem, add=True)
                  for j in range(_CH)]
        for cp in copies:
          cp.wait()
        return c

      lax.fori_loop(0, nb // _CH, cstep, 0)
      plsc.subcore_barrier()
      pltpu.sync_copy(acc.at[pl.ds(base, rows_per_tile)],
                      cnt_hbm.at[pl.ds(obase, rows_per_tile)])
      zero_acc()
    plsc.subcore_barrier()

    def chunk_step(k, c):
      row0 = wid * nb + k * _CH
      pltpu.sync_copy(src_hbm.at[pl.ds(row0, _CH)], srcv)
      pltpu.sync_copy(dst_hbm.at[pl.ds(row0, _CH)], dstv)
      gs = [None] * _CH
      ss = [None] * _CH
      for t in range(_NK - 1):
        gs[t] = pltpu.async_copy(h_hbm.at[srcv.at[t]], bufs[t % _NK], gsem)
      for j in range(_CH):
        gs[j].wait()
        ss[j] = pltpu.async_copy(bufs[j % _NK], acc.at[dstv.at[j]], ssem,
                                 add=True)
        nxt = j + _NK - 1
        if nxt < _CH:
          if j >= 1:
            ss[j - 1].wait()
          gs[nxt] = pltpu.async_copy(h_hbm.at[srcv.at[nxt]], bufs[nxt % _NK],
                                     gsem)
      for j in range(max(0, _CH - _NK), _CH):
        ss[j].wait()
      return c

    lax.fori_loop(0, nb // _CH, chunk_step, 0)
    plsc.subcore_barrier()

    pltpu.sync_copy(acc.at[pl.ds(base, rows_per_tile)],
                    sum_hbm.at[pl.ds(obase, rows_per_tile)])

  return pl.kernel(body, out_type=tuple(outs), scratch_types=scratch,
                   mesh=mesh)


def _dense1_body(a0, a1, c0, c1, x, wn, wr, b, o):
  cnt = c0[...] + c1[...]
  inv = 1.0 / jnp.maximum(cnt, 1.0)
  mean = (a0[...] + a1[...]) * inv
  acc = jnp.dot(mean, wn[...], preferred_element_type=jnp.float32)
  acc += jnp.dot(x[...], wr[...], preferred_element_type=jnp.float32)
  o[...] = jnp.maximum(acc + b[...], 0.0)


def _dense2_body(a0, a1, c0, c1, h, mk, wn, wr, b, wf1, bf1, wf2, bf2, o):
  cnt = c0[...] + c1[...]
  inv = 1.0 / jnp.maximum(cnt, 1.0)
  mean = (a0[...] + a1[...]) * inv
  acc = jnp.dot(mean, wn[...], preferred_element_type=jnp.float32)
  acc += jnp.dot(h[...], wr[...], preferred_element_type=jnp.float32)
  h2 = jnp.maximum(acc + b[...], 0.0) * mk[...]
  z = jnp.maximum(
      jnp.dot(h2, wf1[...], preferred_element_type=jnp.float32) + bf1[...],
      0.0)
  o[...] = jnp.dot(z, wf2[...], preferred_element_type=jnp.float32) + bf2[...]


def _row_spec(r, w):
  return pl.BlockSpec((r, w), lambda i: (i, 0))


def _full_spec(s0, s1):
  return pl.BlockSpec((s0, s1), lambda i: (0, 0))


def kernel(x, edge_index, mask, Wn1, Wr1, b1, Wn2, Wr2, b2, Wf1, bf1,
           Wf2, bf2):
  n, d = x.shape
  e = edge_index.shape[1]
  h1w = Wn1.shape[1]
  h2w = Wn2.shape[1]
  lw = Wf1.shape[1]
  c = Wf2.shape[1]

  ep = -(-e // (_NW * _BB * _CH)) * (_NW * _BB * _CH)
  nb = ep // (_NW * _BB)
  pad = ep - e
  src = edge_index[0]
  dst = edge_index[1]
  if pad:
    src = jnp.concatenate([src, jnp.zeros((pad,), jnp.int32)])
    dst = jnp.concatenate([dst, jnp.full((pad,), n, jnp.int32)])
  src = src.reshape(_NW * nb, _BB)
  dst = dst.reshape(_NW * nb, _BB)
  acc_rows = -(-(n + 1) // 128) * 128

  sc_pass1 = _make_sc_scatter(n, d, acc_rows, nb, True)
  sums1, cnts = sc_pass1(x, src, dst)
  a0, a1 = sums1[:n], sums1[acc_rows:acc_rows + n]
  c0, c1 = cnts[:n, 0:1], cnts[acc_rows:acc_rows + n, 0:1]

  r = 1000
  grid = (n // r,)
  h1 = pl.pallas_call(
      _dense1_body,
      grid=grid,
      in_specs=[
          _row_spec(r, d), _row_spec(r, d),
          _row_spec(r, 1), _row_spec(r, 1),
          _row_spec(r, d),
          _full_spec(d, h1w), _full_spec(d, h1w), _full_spec(1, h1w),
      ],
      out_specs=_row_spec(r, h1w),
      out_shape=jax.ShapeDtypeStruct((n, h1w), jnp.float32),
  )(a0, a1, c0, c1, x, Wn1, Wr1, b1.reshape(1, h1w))

  sc_pass2 = _make_sc_scatter(n, h1w, acc_rows, nb, False)
  (sums2,) = sc_pass2(h1, src, dst)
  a0b, a1b = sums2[:n], sums2[acc_rows:acc_rows + n]

  maskf = mask.astype(jnp.float32).reshape(n, 1)
  out = pl.pallas_call(
      _dense2_body,
      grid=grid,
      in_specs=[
          _row_spec(r, h1w), _row_spec(r, h1w),
          _row_spec(r, 1), _row_spec(r, 1),
          _row_spec(r, h1w), _row_spec(r, 1),
          _full_spec(h1w, h2w), _full_spec(h1w, h2w), _full_spec(1, h2w),
          _full_spec(h2w, lw), _full_spec(1, lw),
          _full_spec(lw, c), _full_spec(1, c),
      ],
      out_specs=_row_spec(r, c),
      out_shape=jax.ShapeDtypeStruct((n, c), jnp.float32),
  )(a0b, a1b, c0, c1, h1, maskf, Wn2, Wr2, b2.reshape(1, h2w),
    Wf1, bf1.reshape(1, lw), Wf2, bf2.reshape(1, c))
  return out


# speedup vs baseline: 1.1453x; 1.1453x over previous
import functools

import jax
import jax.numpy as jnp
from jax import lax
from jax.experimental import pallas as pl
from jax.experimental.pallas import tpu as pltpu
from jax.experimental.pallas import tpu_sc as plsc

_NC = 2
_NS = 16
_NW = _NC * _NS
_BB = 64
_CH = 32
_NK = 4


def _make_sc_scatter(n_rows, d, acc_rows, nb, with_count):
  rows_per_tile = acc_rows // _NS
  mesh = plsc.VectorSubcoreMesh(core_axis_name="c", subcore_axis_name="s")
  outs = [jax.ShapeDtypeStruct((_NC * acc_rows, d), jnp.float32)]
  scratch = [
      pltpu.VMEM((_CH, _BB), jnp.int32),
      pltpu.VMEM((_CH, _BB), jnp.int32),
  ] + [
      pltpu.VMEM((_BB, d), jnp.float32)
      for _ in range(_NK)
  ] + [
      pltpu.VMEM((8, d), jnp.float32),
      pltpu.VMEM_SHARED((acc_rows, d), jnp.float32),
      pltpu.SemaphoreType.DMA,
      pltpu.SemaphoreType.DMA,
  ]
  if with_count:
    outs.append(jax.ShapeDtypeStruct((_NC * acc_rows, d), jnp.float32))

  def body(h_hbm, src_hbm, dst_hbm, *rest):
    if with_count:
      sum_hbm, cnt_hbm, srcv, dstv = rest[:4]
      bufs = rest[4:4 + _NK]
      zbuf, acc, gsem, ssem = rest[4 + _NK:]
    else:
      sum_hbm, srcv, dstv = rest[:3]
      bufs = rest[3:3 + _NK]
      zbuf, acc, gsem, ssem = rest[3 + _NK:]
    buf0 = bufs[0]
    cid = lax.axis_index("c")
    sid = lax.axis_index("s")
    wid = sid * _NC + cid
    zv = jnp.zeros((16,), jnp.float32)
    for r in range(8):
      for c8 in range(d // 16):
        zbuf[r, pl.ds(c8 * 16, 16)] = zv
    base = sid * rows_per_tile
    obase = cid * acc_rows + base

    def zero_acc():
      def zstep(j, c):
        pltpu.sync_copy(zbuf, acc.at[pl.ds(base + j * 8, 8)])
        return c
      lax.fori_loop(0, rows_per_tile // 8, zstep, 0)

    zero_acc()
    if with_count:
      ov = jnp.ones((16,), jnp.float32)
      for r in range(_BB):
        for c8 in range(d // 16):
          buf0[r, pl.ds(c8 * 16, 16)] = ov
      plsc.subcore_barrier()

      def cstep(k, c):
        row0 = wid * nb + k * _CH
        pltpu.sync_copy(dst_hbm.at[pl.ds(row0, _CH)], dstv)
        copies = [pltpu.async_copy(buf0, acc.at[dstv.at[j]], ssem, add=True)
                  for j in range(_CH)]
        for cp in copies:
          cp.wait()
        return c

      lax.fori_loop(0, nb // _CH, cstep, 0)
      plsc.subcore_barrier()
      pltpu.sync_copy(acc.at[pl.ds(base, rows_per_tile)],
                      cnt_hbm.at[pl.ds(obase, rows_per_tile)])
      zero_acc()
    plsc.subcore_barrier()

    def chunk_step(k, c):
      row0 = wid * nb + k * _CH
      pltpu.sync_copy(src_hbm.at[pl.ds(row0, _CH)], srcv)
      pltpu.sync_copy(dst_hbm.at[pl.ds(row0, _CH)], dstv)
      gs = [None] * _CH
      ss = [None] * _CH
      for t in range(_NK - 1):
        gs[t] = pltpu.async_copy(h_hbm.at[srcv.at[t]], bufs[t % _NK], gsem)
      for j in range(_CH):
        gs[j].wait()
        ss[j] = pltpu.async_copy(bufs[j % _NK], acc.at[dstv.at[j]], ssem,
                                 add=True)
        nxt = j + _NK - 1
        if nxt < _CH:
          if j >= 1:
            ss[j - 1].wait()
          gs[nxt] = pltpu.async_copy(h_hbm.at[srcv.at[nxt]], bufs[nxt % _NK],
                                     gsem)
      for j in range(max(0, _CH - _NK), _CH):
        ss[j].wait()
      return c

    lax.fori_loop(0, nb // _CH, chunk_step, 0)
    plsc.subcore_barrier()

    pltpu.sync_copy(acc.at[pl.ds(base, rows_per_tile)],
                    sum_hbm.at[pl.ds(obase, rows_per_tile)])

  return pl.kernel(body, out_type=tuple(outs), scratch_types=scratch,
                   mesh=mesh)


def _dense1_body(a0, a1, c0, c1, x, wn, wr, b, o):
  cnt = c0[...] + c1[...]
  inv = 1.0 / jnp.maximum(cnt, 1.0)
  mean = (a0[...] + a1[...]) * inv
  acc = jnp.dot(mean, wn[...], preferred_element_type=jnp.float32)
  acc += jnp.dot(x[...], wr[...], preferred_element_type=jnp.float32)
  o[...] = jnp.maximum(acc + b[...], 0.0)


def _dense2_body(a0, a1, c0, c1, h, mk, wn, wr, b, wf1, bf1, wf2, bf2, o):
  cnt = c0[...] + c1[...]
  inv = 1.0 / jnp.maximum(cnt, 1.0)
  mean = (a0[...] + a1[...]) * inv
  acc = jnp.dot(mean, wn[...], preferred_element_type=jnp.float32)
  acc += jnp.dot(h[...], wr[...], preferred_element_type=jnp.float32)
  h2 = jnp.maximum(acc + b[...], 0.0) * mk[...]
  z = jnp.maximum(
      jnp.dot(h2, wf1[...], preferred_element_type=jnp.float32) + bf1[...],
      0.0)
  o[...] = jnp.dot(z, wf2[...], preferred_element_type=jnp.float32) + bf2[...]


def _row_spec(r, w):
  return pl.BlockSpec((r, w), lambda i: (i, 0))


def _full_spec(s0, s1):
  return pl.BlockSpec((s0, s1), lambda i: (0, 0))


def kernel(x, edge_index, mask, Wn1, Wr1, b1, Wn2, Wr2, b2, Wf1, bf1,
           Wf2, bf2):
  n, d = x.shape
  e = edge_index.shape[1]
  h1w = Wn1.shape[1]
  h2w = Wn2.shape[1]
  lw = Wf1.shape[1]
  c = Wf2.shape[1]

  ep = -(-e // (_NW * _BB * _CH)) * (_NW * _BB * _CH)
  nb = ep // (_NW * _BB)
  pad = ep - e
  src = edge_index[0]
  dst = edge_index[1]
  if pad:
    src = jnp.concatenate([src, jnp.zeros((pad,), jnp.int32)])
    dst = jnp.concatenate([dst, jnp.full((pad,), n, jnp.int32)])
  src = src.reshape(_NW * nb, _BB)
  dst = dst.reshape(_NW * nb, _BB)
  acc_rows = -(-(n + 1) // 128) * 128

  sc_pass1 = _make_sc_scatter(n, d, acc_rows, nb, True)
  sums1, cnts = sc_pass1(x, src, dst)
  a0, a1 = sums1[:n], sums1[acc_rows:acc_rows + n]
  c0, c1 = cnts[:n, 0:1], cnts[acc_rows:acc_rows + n, 0:1]

  r = 1000
  grid = (n // r,)
  h1 = pl.pallas_call(
      _dense1_body,
      grid=grid,
      in_specs=[
          _row_spec(r, d), _row_spec(r, d),
          _row_spec(r, 1), _row_spec(r, 1),
          _row_spec(r, d),
          _full_spec(d, h1w), _full_spec(d, h1w), _full_spec(1, h1w),
      ],
      out_specs=_row_spec(r, h1w),
      out_shape=jax.ShapeDtypeStruct((n, h1w), jnp.float32),
  )(a0, a1, c0, c1, x, Wn1, Wr1, b1.reshape(1, h1w))

  sc_pass2 = _make_sc_scatter(n, h1w, acc_rows, nb, False)
  (sums2,) = sc_pass2(h1, src, dst)
  a0b, a1b = sums2[:n], sums2[acc_rows:acc_rows + n]

  maskf = mask.astype(jnp.float32).reshape(n, 1)
  out = pl.pallas_call(
      _dense2_body,
      grid=grid,
      in_specs=[
          _row_spec(r, h1w), _row_spec(r, h1w),
          _row_spec(r, 1), _row_spec(r, 1),
          _row_spec(r, h1w), _row_spec(r, 1),
          _full_spec(h1w, h2w), _full_spec(h1w, h2w), _full_spec(1, h2w),
          _full_spec(h2w, lw), _full_spec(1, lw),
          _full_spec(lw, c), _full_spec(1, c),
      ],
      out_specs=_row_spec(r, c),
      out_shape=jax.ShapeDtypeStruct((n, c), jnp.float32),
  )(a0b, a1b, c0, c1, h1, maskf, Wn2, Wr2, b2.reshape(1, h2w),
    Wf1, bf1.reshape(1, lw), Wf2, bf2.reshape(1, c))
  return out


# speedup vs baseline: 1.1464x; 1.0009x over previous
import functools

import jax
import jax.numpy as jnp
from jax import lax
from jax.experimental import pallas as pl
from jax.experimental.pallas import tpu as pltpu
from jax.experimental.pallas import tpu_sc as plsc

_NC = 2
_NS = 16
_NW = _NC * _NS
_BB = 64
_CH = 32
_NK = 5


def _make_sc_scatter(n_rows, d, acc_rows, nb, with_count):
  rows_per_tile = acc_rows // _NS
  mesh = plsc.VectorSubcoreMesh(core_axis_name="c", subcore_axis_name="s")
  outs = [jax.ShapeDtypeStruct((_NC * acc_rows, d), jnp.float32)]
  scratch = [
      pltpu.VMEM((_CH, _BB), jnp.int32),
      pltpu.VMEM((_CH, _BB), jnp.int32),
  ] + [
      pltpu.VMEM((_BB, d), jnp.float32)
      for _ in range(_NK)
  ] + [
      pltpu.VMEM((8, d), jnp.float32),
      pltpu.VMEM_SHARED((acc_rows, d), jnp.float32),
      pltpu.SemaphoreType.DMA,
      pltpu.SemaphoreType.DMA,
  ]
  if with_count:
    outs.append(jax.ShapeDtypeStruct((_NC * acc_rows, d), jnp.float32))

  def body(h_hbm, src_hbm, dst_hbm, *rest):
    if with_count:
      sum_hbm, cnt_hbm, srcv, dstv = rest[:4]
      bufs = rest[4:4 + _NK]
      zbuf, acc, gsem, ssem = rest[4 + _NK:]
    else:
      sum_hbm, srcv, dstv = rest[:3]
      bufs = rest[3:3 + _NK]
      zbuf, acc, gsem, ssem = rest[3 + _NK:]
    buf0 = bufs[0]
    cid = lax.axis_index("c")
    sid = lax.axis_index("s")
    wid = sid * _NC + cid
    zv = jnp.zeros((16,), jnp.float32)
    for r in range(8):
      for c8 in range(d // 16):
        zbuf[r, pl.ds(c8 * 16, 16)] = zv
    base = sid * rows_per_tile
    obase = cid * acc_rows + base

    def zero_acc():
      def zstep(j, c):
        pltpu.sync_copy(zbuf, acc.at[pl.ds(base + j * 8, 8)])
        return c
      lax.fori_loop(0, rows_per_tile // 8, zstep, 0)

    zero_acc()
    if with_count:
      ov = jnp.ones((16,), jnp.float32)
      for r in range(_BB):
        for c8 in range(d // 16):
          buf0[r, pl.ds(c8 * 16, 16)] = ov
      plsc.subcore_barrier()

      def cstep(k, c):
        row0 = wid * nb + k * _CH
        pltpu.sync_copy(dst_hbm.at[pl.ds(row0, _CH)], dstv)
        copies = [pltpu.async_copy(buf0, acc.at[dstv.at[j]], ssem, add=True)
                  for j in range(_CH)]
        for cp in copies:
          cp.wait()
        return c

      lax.fori_loop(0, nb // _CH, cstep, 0)
      plsc.subcore_barrier()
      pltpu.sync_copy(acc.at[pl.ds(base, rows_per_tile)],
                      cnt_hbm.at[pl.ds(obase, rows_per_tile)])
      zero_acc()
    plsc.subcore_barrier()

    def chunk_step(k, c):
      row0 = wid * nb + k * _CH
      pltpu.sync_copy(src_hbm.at[pl.ds(row0, _CH)], srcv)
      pltpu.sync_copy(dst_hbm.at[pl.ds(row0, _CH)], dstv)
      gs = [None] * _CH
      ss = [None] * _CH
      for t in range(_NK - 1):
        gs[t] = pltpu.async_copy(h_hbm.at[srcv.at[t]], bufs[t % _NK], gsem)
      for j in range(_CH):
        gs[j].wait()
        ss[j] = pltpu.async_copy(bufs[j % _NK], acc.at[dstv.at[j]], ssem,
                                 add=True)
        nxt = j + _NK - 1
        if nxt < _CH:
          if j >= 1:
            ss[j - 1].wait()
          gs[nxt] = pltpu.async_copy(h_hbm.at[srcv.at[nxt]], bufs[nxt % _NK],
                                     gsem)
      for j in range(max(0, _CH - _NK), _CH):
        ss[j].wait()
      return c

    lax.fori_loop(0, nb // _CH, chunk_step, 0)
    plsc.subcore_barrier()

    pltpu.sync_copy(acc.at[pl.ds(base, rows_per_tile)],
                    sum_hbm.at[pl.ds(obase, rows_per_tile)])

  return pl.kernel(body, out_type=tuple(outs), scratch_types=scratch,
                   mesh=mesh)


def _dense1_body(a0, a1, c0, c1, x, wn, wr, b, o):
  cnt = c0[...] + c1[...]
  inv = 1.0 / jnp.maximum(cnt, 1.0)
  mean = (a0[...] + a1[...]) * inv
  acc = jnp.dot(mean, wn[...], preferred_element_type=jnp.float32)
  acc += jnp.dot(x[...], wr[...], preferred_element_type=jnp.float32)
  o[...] = jnp.maximum(acc + b[...], 0.0)


def _dense2_body(a0, a1, c0, c1, h, mk, wn, wr, b, wf1, bf1, wf2, bf2, o):
  cnt = c0[...] + c1[...]
  inv = 1.0 / jnp.maximum(cnt, 1.0)
  mean = (a0[...] + a1[...]) * inv
  acc = jnp.dot(mean, wn[...], preferred_element_type=jnp.float32)
  acc += jnp.dot(h[...], wr[...], preferred_element_type=jnp.float32)
  h2 = jnp.maximum(acc + b[...], 0.0) * mk[...]
  z = jnp.maximum(
      jnp.dot(h2, wf1[...], preferred_element_type=jnp.float32) + bf1[...],
      0.0)
  o[...] = jnp.dot(z, wf2[...], preferred_element_type=jnp.float32) + bf2[...]


def _row_spec(r, w):
  return pl.BlockSpec((r, w), lambda i: (i, 0))


def _full_spec(s0, s1):
  return pl.BlockSpec((s0, s1), lambda i: (0, 0))


def kernel(x, edge_index, mask, Wn1, Wr1, b1, Wn2, Wr2, b2, Wf1, bf1,
           Wf2, bf2):
  n, d = x.shape
  e = edge_index.shape[1]
  h1w = Wn1.shape[1]
  h2w = Wn2.shape[1]
  lw = Wf1.shape[1]
  c = Wf2.shape[1]

  ep = -(-e // (_NW * _BB * _CH)) * (_NW * _BB * _CH)
  nb = ep // (_NW * _BB)
  pad = ep - e
  src = edge_index[0]
  dst = edge_index[1]
  if pad:
    src = jnp.concatenate([src, jnp.zeros((pad,), jnp.int32)])
    dst = jnp.concatenate([dst, jnp.full((pad,), n, jnp.int32)])
  src = src.reshape(_NW * nb, _BB)
  dst = dst.reshape(_NW * nb, _BB)
  acc_rows = -(-(n + 1) // 128) * 128

  sc_pass1 = _make_sc_scatter(n, d, acc_rows, nb, True)
  sums1, cnts = sc_pass1(x, src, dst)
  a0, a1 = sums1[:n], sums1[acc_rows:acc_rows + n]
  c0, c1 = cnts[:n, 0:1], cnts[acc_rows:acc_rows + n, 0:1]

  r = 1000
  grid = (n // r,)
  h1 = pl.pallas_call(
      _dense1_body,
      grid=grid,
      in_specs=[
          _row_spec(r, d), _row_spec(r, d),
          _row_spec(r, 1), _row_spec(r, 1),
          _row_spec(r, d),
          _full_spec(d, h1w), _full_spec(d, h1w), _full_spec(1, h1w),
      ],
      out_specs=_row_spec(r, h1w),
      out_shape=jax.ShapeDtypeStruct((n, h1w), jnp.float32),
  )(a0, a1, c0, c1, x, Wn1, Wr1, b1.reshape(1, h1w))

  sc_pass2 = _make_sc_scatter(n, h1w, acc_rows, nb, False)
  (sums2,) = sc_pass2(h1, src, dst)
  a0b, a1b = sums2[:n], sums2[acc_rows:acc_rows + n]

  maskf = mask.astype(jnp.float32).reshape(n, 1)
  out = pl.pallas_call(
      _dense2_body,
      grid=grid,
      in_specs=[
          _row_spec(r, h1w), _row_spec(r, h1w),
          _row_spec(r, 1), _row_spec(r, 1),
          _row_spec(r, h1w), _row_spec(r, 1),
          _full_spec(h1w, h2w), _full_spec(h1w, h2w), _full_spec(1, h2w),
          _full_spec(h2w, lw), _full_spec(1, lw),
          _full_spec(lw, c), _full_spec(1, c),
      ],
      out_specs=_row_spec(r, c),
      out_shape=jax.ShapeDtypeStruct((n, c), jnp.float32),
  )(a0b, a1b, c0, c1, h1, maskf, Wn2, Wr2, b2.reshape(1, h2w),
    Wf1, bf1.reshape(1, lw), Wf2, bf2.reshape(1, c))
  return out
